# baseline (speedup 1.0000x reference)
import functools

import jax
import jax.numpy as jnp
from jax import lax
from jax.experimental import pallas as pl
from jax.experimental.pallas import tpu as pltpu
from jax.experimental.pallas import tpu_sc as plsc

_NV = 10000
_NC = 42000
_K = 3
_B = 128
_NSUB = 16
_ROWS = 2640
_ROWS_LAST = _NC - 15 * _ROWS
_CHUNK = 240
_GPC = _CHUNK // 16
_NCH = _ROWS // _CHUNK
_NCH_LAST = _ROWS_LAST // _CHUNK
_THRESH = float(_NC - 1)


def _sat_body(emb_hbm, idx_hbm, w_hbm, out_hbm,
              table_v, idx_s, w_s, part_v, part_sh, all_v, out_v,
              sem_i, sem_w):
    cid = lax.axis_index("c")
    sid = lax.axis_index("s")
    row0 = sid * _ROWS
    pltpu.sync_copy(emb_hbm.at[0], table_v)

    lanes = lax.iota(jnp.int32, 16)
    cols = [jnp.full((16,), j, jnp.int32) for j in range(_K)]
    two = jnp.full((16,), float(_K - 1), jnp.float32)

    def chunk_body(c, acc):
        r = pl.multiple_of(row0 + c * _CHUNK, 8)
        cp_i = pltpu.make_async_copy(idx_hbm.at[pl.ds(r, _CHUNK)], idx_s, sem_i)
        cp_w = pltpu.make_async_copy(w_hbm.at[pl.ds(r, _CHUNK)], w_s, sem_w)
        cp_i.start()
        cp_w.start()
        cp_i.wait()
        cp_w.wait()
        for g in range(_GPC):
            rows = lanes + g * 16
            pre = two
            for j in range(_K):
                lit = plsc.load_gather(idx_s, [rows, cols[j]])
                ev = plsc.load_gather(table_v, [lit])
                wv = plsc.load_gather(w_s, [rows, cols[j]])
                pre = pre + wv * jnp.sign(ev)
            acc = acc + jnp.sign(pre)
        return acc

    nchunks = jnp.where(sid == 15, _NCH_LAST, _NCH)
    acc = lax.fori_loop(0, nchunks, chunk_body, jnp.zeros((16,), jnp.float32))

    part_v[...] = acc
    pltpu.sync_copy(part_v, part_sh.at[sid])
    plsc.subcore_barrier()

    @pl.when(jnp.logical_and(cid == 0, sid == 0))
    def _finish():
        pltpu.sync_copy(part_sh, all_v)
        tot = all_v[0]
        for r in range(1, _NSUB):
            tot = tot + all_v[r]
        total = jnp.sum(tot)
        res = jnp.sign(total - _THRESH)
        resv = jnp.full((16,), res, jnp.float32)
        for k in range(_B // 16):
            out_v[pl.ds(k * 16, 16)] = resv
        pltpu.sync_copy(out_v, out_hbm)


@functools.lru_cache(maxsize=1)
def _build():
    mesh = plsc.VectorSubcoreMesh(
        core_axis_name="c", subcore_axis_name="s",
        num_cores=2, num_subcores=_NSUB,
    )
    return pl.kernel(
        _sat_body,
        out_type=jax.ShapeDtypeStruct((_B,), jnp.float32),
        mesh=mesh,
        compiler_params=pltpu.CompilerParams(
            needs_layout_passes=False, use_tc_tiling_on_sc=False
        ),
        scratch_types=[
            pltpu.VMEM((_NV,), jnp.float32),
            pltpu.VMEM((_CHUNK, _K), jnp.int32),
            pltpu.VMEM((_CHUNK, _K), jnp.float32),
            pltpu.VMEM((16,), jnp.float32),
            pltpu.VMEM_SHARED((_NSUB, 16), jnp.float32),
            pltpu.VMEM((_NSUB, 16), jnp.float32),
            pltpu.VMEM((_B,), jnp.float32),
            pltpu.SemaphoreType.DMA,
            pltpu.SemaphoreType.DMA,
        ],
    )


def kernel(input, emb_weight, or_weight, clause_idx):
    del input
    return _build()(emb_weight, clause_idx, or_weight)


# speedup vs baseline: 2.1971x; 2.1971x over previous
import functools

import jax
import jax.numpy as jnp
from jax import lax
from jax.experimental import pallas as pl
from jax.experimental.pallas import tpu as pltpu
from jax.experimental.pallas import tpu_sc as plsc

_NV = 10000
_NC = 42000
_K = 3
_B = 128
_NSUB = 16
_CPW = _NC // _NSUB
_LPW = _CPW * _K
_FULL = _CPW // 16
_REM = _CPW - _FULL * 16
_WIN = 7888
_WIN_LAST = 7880
_BUF = 7936
_THRESH = float(_NC - 1 + _NSUB * (16 - _REM))


def _sat_body(emb_hbm, lit_hbm, out_hbm,
              table_v, lit_v, part_v, part_sh, all_v, out_v):
    cid = lax.axis_index("c")
    sid = lax.axis_index("s")
    start = sid * _LPW
    astart = pl.multiple_of((start // 8) * 8, 8)
    delta = start - astart
    pltpu.sync_copy(emb_hbm.at[0], table_v)

    @pl.when(sid < 15)
    def _copy_most():
        pltpu.sync_copy(lit_hbm.at[pl.ds(astart, _WIN)],
                        lit_v.at[pl.ds(0, _WIN)])

    @pl.when(sid == 15)
    def _copy_last():
        pltpu.sync_copy(lit_hbm.at[pl.ds(astart, _WIN_LAST)],
                        lit_v.at[pl.ds(0, _WIN_LAST)])

    lanes = lax.iota(jnp.int32, 16)
    lanes3 = lanes * _K
    two = jnp.full((16,), float(_K - 1), jnp.float32)

    def clause_group(base, pre):
        for j in range(_K):
            p = plsc.load_gather(lit_v, [lanes3 + (base + j)])
            lit = lax.shift_right_logical(p, 1)
            neg = (p & 1) == 1
            ev = plsc.load_gather(table_v, [lit])
            sv = jnp.sign(ev)
            pre = pre + jnp.where(neg, -sv, sv)
        return pre

    def body(i, acc):
        pre = clause_group(delta + i * (16 * _K), two)
        return acc + jnp.sign(pre)

    acc = lax.fori_loop(0, _FULL, body, jnp.zeros((16,), jnp.float32))

    valid = lanes < _REM
    base = delta + _FULL * (16 * _K)
    pre = two
    for j in range(_K):
        p = plsc.load_gather(lit_v, [lanes3 + (base + j)])
        lit = jnp.where(valid, lax.shift_right_logical(p, 1), 0)
        neg = (p & 1) == 1
        ev = plsc.load_gather(table_v, [lit])
        sv = jnp.sign(ev)
        pre = pre + jnp.where(valid, jnp.where(neg, -sv, sv), 0.0)
    acc = acc + jnp.sign(pre)

    part_v[...] = acc
    pltpu.sync_copy(part_v, part_sh.at[sid])
    plsc.subcore_barrier()

    @pl.when(jnp.logical_and(cid == 0, sid == 0))
    def _finish():
        pltpu.sync_copy(part_sh, all_v)
        tot = all_v[0]
        for r in range(1, _NSUB):
            tot = tot + all_v[r]
        total = jnp.sum(tot)
        res = jnp.sign(total - _THRESH)
        resv = jnp.full((16,), res, jnp.float32)
        for k in range(_B // 16):
            out_v[pl.ds(k * 16, 16)] = resv
        pltpu.sync_copy(out_v, out_hbm)


@functools.lru_cache(maxsize=1)
def _build():
    mesh = plsc.VectorSubcoreMesh(
        core_axis_name="c", subcore_axis_name="s",
        num_cores=2, num_subcores=_NSUB,
    )
    return pl.kernel(
        _sat_body,
        out_type=jax.ShapeDtypeStruct((_B,), jnp.float32),
        mesh=mesh,
        compiler_params=pltpu.CompilerParams(needs_layout_passes=False),
        scratch_types=[
            pltpu.VMEM((_NV,), jnp.float32),
            pltpu.VMEM((_BUF,), jnp.int32),
            pltpu.VMEM((16,), jnp.float32),
            pltpu.VMEM_SHARED((_NSUB, 16), jnp.float32),
            pltpu.VMEM((_NSUB, 16), jnp.float32),
            pltpu.VMEM((_B,), jnp.float32),
        ],
    )


def kernel(input, emb_weight, or_weight, clause_idx):
    del input
    packed = lax.shift_left(clause_idx, 1) | (or_weight < 0).astype(jnp.int32)
    return _build()(emb_weight, packed.reshape(-1))


# speedup vs baseline: 4.3226x; 1.9674x over previous
import functools

import jax
import jax.numpy as jnp
from jax import lax
from jax.experimental import pallas as pl
from jax.experimental.pallas import tpu as pltpu
from jax.experimental.pallas import tpu_sc as plsc

_NV = 10000
_NC = 42000
_K = 3
_B = 128
_NSUB = 16
_CPW = _NC // _NSUB
_FULL = _CPW // 16
_REM = _CPW - _FULL * 16
_WIN = 2640
_WIN_LAST = 2632
_BUF = 2648
_THRESH = float(_NC - 1 + _NSUB * (16 - _REM))


def _sat_body(emb_hbm, lit_hbm, out_hbm,
              table_v, l0, l1, l2, part_v, part_sh, all_v, out_v):
    cid = lax.axis_index("c")
    sid = lax.axis_index("s")
    start = sid * _CPW
    astart = (start // 8) * 8
    delta = start - astart
    pltpu.sync_copy(emb_hbm.at[0], table_v)
    bufs = (l0, l1, l2)

    @pl.when(sid < 15)
    def _copy_most():
        for j in range(_K):
            off = pl.multiple_of(j * _NC + astart, 8)
            pltpu.sync_copy(lit_hbm.at[pl.ds(off, _WIN)],
                            bufs[j].at[pl.ds(0, _WIN)])

    @pl.when(sid == 15)
    def _copy_last():
        for j in range(_K):
            off = pl.multiple_of(j * _NC + astart, 8)
            pltpu.sync_copy(lit_hbm.at[pl.ds(off, _WIN_LAST)],
                            bufs[j].at[pl.ds(0, _WIN_LAST)])

    lanes = lax.iota(jnp.int32, 16)
    two = jnp.full((16,), float(_K - 1), jnp.float32)

    def body(i, acc):
        base = delta + i * 16
        pre = two
        for j in range(_K):
            p = bufs[j][pl.ds(base, 16)]
            lit = lax.shift_right_logical(p, 1)
            neg = (p & 1) == 1
            ev = plsc.load_gather(table_v, [lit])
            sv = jnp.sign(ev)
            pre = pre + jnp.where(neg, -sv, sv)
        return acc + jnp.sign(pre)

    acc = lax.fori_loop(0, _FULL, body, jnp.zeros((16,), jnp.float32))

    valid = lanes < _REM
    base = delta + _FULL * 16
    pre = two
    for j in range(_K):
        p = bufs[j][pl.ds(base, 16)]
        lit = jnp.where(valid, lax.shift_right_logical(p, 1), 0)
        neg = (p & 1) == 1
        ev = plsc.load_gather(table_v, [lit])
        sv = jnp.sign(ev)
        pre = pre + jnp.where(valid, jnp.where(neg, -sv, sv), 0.0)
    acc = acc + jnp.sign(pre)

    part_v[...] = acc
    pltpu.sync_copy(part_v, part_sh.at[sid])
    plsc.subcore_barrier()

    @pl.when(jnp.logical_and(cid == 0, sid == 0))
    def _finish():
        pltpu.sync_copy(part_sh, all_v)
        tot = all_v[0]
        for r in range(1, _NSUB):
            tot = tot + all_v[r]
        total = jnp.sum(tot)
        res = jnp.sign(total - _THRESH)
        resv = jnp.full((16,), res, jnp.float32)
        for k in range(_B // 16):
            out_v[pl.ds(k * 16, 16)] = resv
        pltpu.sync_copy(out_v, out_hbm)


@functools.lru_cache(maxsize=1)
def _build():
    mesh = plsc.VectorSubcoreMesh(
        core_axis_name="c", subcore_axis_name="s",
        num_cores=2, num_subcores=_NSUB,
    )
    return pl.kernel(
        _sat_body,
        out_type=jax.ShapeDtypeStruct((_B,), jnp.float32),
        mesh=mesh,
        compiler_params=pltpu.CompilerParams(needs_layout_passes=False),
        scratch_types=[
            pltpu.VMEM((_NV,), jnp.float32),
            pltpu.VMEM((_BUF,), jnp.int32),
            pltpu.VMEM((_BUF,), jnp.int32),
            pltpu.VMEM((_BUF,), jnp.int32),
            pltpu.VMEM((16,), jnp.float32),
            pltpu.VMEM_SHARED((_NSUB, 16), jnp.float32),
            pltpu.VMEM((_NSUB, 16), jnp.float32),
            pltpu.VMEM((_B,), jnp.float32),
        ],
    )


def kernel(input, emb_weight, or_weight, clause_idx):
    del input
    packed = lax.shift_left(clause_idx, 1) | (or_weight < 0).astype(jnp.int32)
    return _build()(emb_weight, packed.T.reshape(-1))


# speedup vs baseline: 4.7048x; 1.0884x over previous
import functools

import jax
import jax.numpy as jnp
from jax import lax
from jax.experimental import pallas as pl
from jax.experimental.pallas import tpu as pltpu
from jax.experimental.pallas import tpu_sc as plsc

_NV = 10000
_NC = 42000
_K = 3
_B = 128
_NSUB = 16
_CPW = _NC // _NSUB
_FULL = _CPW // 16
_REM = _CPW - _FULL * 16
_WIN = 2640
_WIN_LAST = 2632
_BUF = 2648
_THRESH = float(_NC - 1 + _NSUB * (16 - _REM))


def _sat_body(emb_hbm, lit_hbm, out_hbm,
              table_v, l0, l1, l2, part_v, part_sh, all_v, out_v,
              sem_t, sem_w):
    cid = lax.axis_index("c")
    sid = lax.axis_index("s")
    start = sid * _CPW
    astart = (start // 8) * 8
    delta = start - astart
    bufs = (l0, l1, l2)
    cp_t = pltpu.make_async_copy(emb_hbm.at[0], table_v, sem_t)
    cp_t.start()

    @pl.when(sid < 15)
    def _copy_most():
        for j in range(_K):
            off = pl.multiple_of(j * _NC + astart, 8)
            pltpu.make_async_copy(lit_hbm.at[pl.ds(off, _WIN)],
                                  bufs[j].at[pl.ds(0, _WIN)], sem_w).start()
        for j in range(_K):
            pltpu.make_async_copy(lit_hbm.at[pl.ds(0, _WIN)],
                                  bufs[j].at[pl.ds(0, _WIN)], sem_w).wait()

    @pl.when(sid == 15)
    def _copy_last():
        for j in range(_K):
            off = pl.multiple_of(j * _NC + astart, 8)
            pltpu.make_async_copy(lit_hbm.at[pl.ds(off, _WIN_LAST)],
                                  bufs[j].at[pl.ds(0, _WIN_LAST)],
                                  sem_w).start()
        for j in range(_K):
            pltpu.make_async_copy(lit_hbm.at[pl.ds(0, _WIN_LAST)],
                                  bufs[j].at[pl.ds(0, _WIN_LAST)],
                                  sem_w).wait()

    cp_t.wait()
    lanes = lax.iota(jnp.int32, 16)

    def body(i, acc):
        base = delta + i * 16
        sat = None
        for j in range(_K):
            p = bufs[j][pl.ds(base, 16)]
            lit = lax.shift_right_logical(p, 1)
            ev = plsc.load_gather(table_v, [lit])
            t = (ev < 0.0) == ((p & 1) == 1)
            sat = t if sat is None else jnp.logical_or(sat, t)
        return acc + jnp.where(sat, 1.0, -1.0)

    acc = lax.fori_loop(0, _FULL, body, jnp.zeros((16,), jnp.float32))

    valid = lanes < _REM
    base = delta + _FULL * 16
    sat = None
    for j in range(_K):
        p = bufs[j][pl.ds(base, 16)]
        lit = jnp.where(valid, lax.shift_right_logical(p, 1), 0)
        ev = plsc.load_gather(table_v, [lit])
        t = (ev < 0.0) == ((p & 1) == 1)
        sat = t if sat is None else jnp.logical_or(sat, t)
    acc = acc + jnp.where(jnp.logical_or(sat, jnp.logical_not(valid)),
                          1.0, -1.0)

    part_v[...] = acc
    pltpu.sync_copy(part_v, part_sh.at[sid])
    plsc.subcore_barrier()

    @pl.when(jnp.logical_and(cid == 0, sid == 0))
    def _finish():
        pltpu.sync_copy(part_sh, all_v)
        tot = all_v[0]
        for r in range(1, _NSUB):
            tot = tot + all_v[r]
        total = jnp.sum(tot)
        res = jnp.sign(total - _THRESH)
        resv = jnp.full((16,), res, jnp.float32)
        for k in range(_B // 16):
            out_v[pl.ds(k * 16, 16)] = resv
        pltpu.sync_copy(out_v, out_hbm)


@functools.lru_cache(maxsize=1)
def _build():
    mesh = plsc.VectorSubcoreMesh(
        core_axis_name="c", subcore_axis_name="s",
        num_cores=2, num_subcores=_NSUB,
    )
    return pl.kernel(
        _sat_body,
        out_type=jax.ShapeDtypeStruct((_B,), jnp.float32),
        mesh=mesh,
        compiler_params=pltpu.CompilerParams(needs_layout_passes=False),
        scratch_types=[
            pltpu.VMEM((_NV,), jnp.float32),
            pltpu.VMEM((_BUF,), jnp.int32),
            pltpu.VMEM((_BUF,), jnp.int32),
            pltpu.VMEM((_BUF,), jnp.int32),
            pltpu.VMEM((16,), jnp.float32),
            pltpu.VMEM_SHARED((_NSUB, 16), jnp.float32),
            pltpu.VMEM((_NSUB, 16), jnp.float32),
            pltpu.VMEM((_B,), jnp.float32),
            pltpu.SemaphoreType.DMA,
            pltpu.SemaphoreType.DMA,
        ],
    )


def kernel(input, emb_weight, or_weight, clause_idx):
    del input
    packed = lax.shift_left(clause_idx, 1) | (or_weight < 0).astype(jnp.int32)
    return _build()(emb_weight, packed.T.reshape(-1))
